# baseline (speedup 1.0000x reference)
import jax
import jax.numpy as jnp
from jax import lax
from jax.experimental import pallas as pl

MS = 14
SP = MS * MS


def _masked_mean_kernel(coords_ref, u_ref, s_ref, o_ref, b_ref):
    p = u_ref.shape[0]
    u = u_ref[...]
    rows = lax.broadcasted_iota(jnp.int32, (p, MS, MS), 1).astype(jnp.float32)
    cols = lax.broadcasted_iota(jnp.int32, (p, MS, MS), 2).astype(jnp.float32)

    def rect(x0, y0, x1, y1):
        m = (rows >= x0[:, :, None]) & (rows < x1[:, :, None]) \
            & (cols >= y0[:, :, None]) & (cols < y1[:, :, None])
        return m.astype(jnp.float32)

    c = coords_ref[...]
    ms = rect(c[:, 0:1], c[:, 4:5], c[:, 1:2], c[:, 5:6])
    mo = rect(c[:, 2:3], c[:, 6:7], c[:, 3:4], c[:, 7:8])
    mb = jnp.maximum(1.0 - ms - mo, 0.0)
    inv = 1.0 / SP
    s_ref[...] = jnp.sum(u * ms[:, None], axis=(2, 3)) * inv
    o_ref[...] = jnp.sum(u * mo[:, None], axis=(2, 3)) * inv
    b_ref[...] = jnp.sum(u * mb[:, None], axis=(2, 3)) * inv


def _fuse_kernel(roi_ref, logits_ref, bboxes_ref, pairs_ref,
                 ss_ref, so_ref, sb_ref,
                 w1a_ref, w1b_ref, w1c_ref, be1_ref, we2_ref, be2_ref,
                 wsr_ref, bsr_ref, wsu_ref, bsu_ref,
                 wor_ref, bor_ref, wou_ref, bou_ref,
                 wbr_ref, bbr_ref, wbu_ref, bbu_ref,
                 wc1a_ref, wc1b_ref, wc1c_ref, bc1_ref, wc2_ref, bc2_ref,
                 wg_ref, bgc_ref, wobj_ref, bobj_ref, wrel_ref, brel_ref,
                 objd_ref, reld_ref):
    f32 = jnp.float32

    def mm(a, b):
        return lax.dot_general(a, b, (((1,), (0,)), ((), ())),
                               preferred_element_type=f32)

    def mm_t(a, b):
        return lax.dot_general(a, b, (((0,), (0,)), ((), ())),
                               preferred_element_type=f32)

    h1 = mm(roi_ref[...], w1a_ref[...]) + mm(logits_ref[...], w1b_ref[...]) \
        + mm(bboxes_ref[...], w1c_ref[...]) + be1_ref[...]
    obj_feats = mm(jnp.maximum(h1, 0.0), we2_ref[...]) + be2_ref[...]

    def gate(s, wr, br, wu, bu):
        a = jax.nn.sigmoid(mm(jnp.maximum(mm(s, wr) + br, 0.0), wu) + bu)
        return s * a

    vs = gate(ss_ref[...], wsr_ref[...], bsr_ref[...], wsu_ref[...], bsu_ref[...])
    vo = gate(so_ref[...], wor_ref[...], bor_ref[...], wou_ref[...], bou_ref[...])
    vb = gate(sb_ref[...], wbr_ref[...], bbr_ref[...], wbu_ref[...], bbu_ref[...])

    rh = jnp.maximum(mm(vs, wc1a_ref[...]) + mm(vo, wc1b_ref[...])
                     + mm(vb, wc1c_ref[...]) + bc1_ref[...], 0.0)
    rel_feats = mm(rh, wc2_ref[...]) + bc2_ref[...]

    n = roi_ref.shape[0]
    m = rel_feats.shape[0]
    pairs = pairs_ref[...]
    obj_ids = lax.broadcasted_iota(jnp.int32, (m, n), 1)
    s_hot = (pairs[:, 0:1] == obj_ids).astype(f32)
    o_hot = (pairs[:, 1:2] == obj_ids).astype(f32)
    so = s_hot + o_hot

    g_obj = mm(obj_feats, wg_ref[...])
    g_rel = mm(rel_feats, wg_ref[...])

    a_oo = mm_t(s_hot, o_hot)
    agg_obj = mm(a_oo, g_obj) + mm_t(so, g_rel) + g_obj
    deg_obj = 1.0 + jnp.sum(a_oo, axis=1, keepdims=True) \
        + jnp.sum(so, axis=0)[:, None]
    h_obj = jnp.maximum(agg_obj / deg_obj + bgc_ref[...], 0.0)

    agg_rel = mm(so, g_obj) + g_rel
    h_rel = jnp.maximum(agg_rel * (1.0 / 3.0) + bgc_ref[...], 0.0)

    out_obj = h_obj + obj_feats
    out_rel = h_rel + rel_feats
    objd_ref[...] = mm(out_obj, wobj_ref[...]) + bobj_ref[...]
    reld_ref[...] = mm(out_rel, wrel_ref[...]) + brel_ref[...]


def kernel(roi_features, obj_logits, bboxes, union_features, rel_pair_idxs,
           We1, be1, We2, be2,
           Wsr, bsr, Wsu, bsu, Wor, bor, Wou, bou, Wbr, bbr, Wbu, bbu,
           Wc1, bc1, Wc2, bc2, Wg, bgc, Wobj, bobj, Wrel, brel):
    f32 = jnp.float32
    n, roi = roi_features.shape
    m, c = union_features.shape[0], union_features.shape[1]
    objc = obj_logits.shape[1]
    relc = Wrel.shape[1]

    sb = bboxes[rel_pair_idxs[:, 0]]
    ob = bboxes[rel_pair_idxs[:, 1]]
    pair_boxes = jnp.concatenate([sb, ob], axis=1)
    union_boxes = jnp.concatenate(
        [jnp.minimum(sb[:, :2], ob[:, :2]), jnp.maximum(sb[:, 2:], ob[:, 2:])], axis=1)
    x = pair_boxes[:, jnp.array([0, 2, 4, 6])] - union_boxes[:, 0:1]
    y = pair_boxes[:, jnp.array([1, 3, 5, 7])] - union_boxes[:, 1:2]
    xr = MS / jnp.maximum(x[:, 1], x[:, 3])
    yr = MS / jnp.maximum(y[:, 1], y[:, 3])
    xp = jnp.clip(jnp.round(x * xr[:, None]), 0, MS)
    yp = jnp.clip(jnp.round(y * yr[:, None]), 0, MS)
    coords = jnp.concatenate([xp, yp], axis=1).astype(f32)

    P = 8
    CB = 128
    grid = (m // P, c // CB)
    ss, so, sbg = pl.pallas_call(
        _masked_mean_kernel,
        grid=grid,
        in_specs=[
            pl.BlockSpec((P, 8), lambda i, j: (i, 0)),
            pl.BlockSpec((P, CB, MS, MS), lambda i, j: (i, j, 0, 0)),
        ],
        out_specs=[
            pl.BlockSpec((P, CB), lambda i, j: (i, j)),
            pl.BlockSpec((P, CB), lambda i, j: (i, j)),
            pl.BlockSpec((P, CB), lambda i, j: (i, j)),
        ],
        out_shape=[jax.ShapeDtypeStruct((m, c), f32)] * 3,
    )(coords, union_features)

    w1a = We1[:roi]
    w1b = We1[roi:roi + objc]
    w1c = We1[roi + objc:]
    wc1a = Wc1[:c]
    wc1b = Wc1[c:2 * c]
    wc1c = Wc1[2 * c:]
    row = lambda v: v.reshape(1, -1)

    obj_dists, rel_dists = pl.pallas_call(
        _fuse_kernel,
        out_shape=[jax.ShapeDtypeStruct((n, objc), f32),
                   jax.ShapeDtypeStruct((m, relc), f32)],
    )(roi_features, obj_logits, bboxes, rel_pair_idxs,
      ss, so, sbg,
      w1a, w1b, w1c, row(be1), We2, row(be2),
      Wsr, row(bsr), Wsu, row(bsu),
      Wor, row(bor), Wou, row(bou),
      Wbr, row(bbr), Wbu, row(bbu),
      wc1a, wc1b, wc1c, row(bc1), Wc2, row(bc2),
      Wg, row(bgc), Wobj, row(bobj), Wrel, row(brel))
    return (obj_dists, rel_dists)


# speedup vs baseline: 4.4477x; 4.4477x over previous
import jax
import jax.numpy as jnp
from jax import lax
from jax.experimental import pallas as pl

MS = 14
SP = MS * MS


def _masked_mean_kernel(coords_ref, u_ref, s_ref, o_ref, b_ref):
    p = u_ref.shape[0]
    u = u_ref[...]
    sidx = lax.broadcasted_iota(jnp.int32, (p, 1, SP), 2).astype(jnp.float32)
    r = jnp.floor((sidx + 0.5) * (1.0 / MS))
    col = sidx - MS * r

    def rect(x0, y0, x1, y1):
        m = (r >= x0[:, :, None]) & (r < x1[:, :, None]) \
            & (col >= y0[:, :, None]) & (col < y1[:, :, None])
        return m.astype(jnp.float32)

    c = coords_ref[...]
    ms = rect(c[:, 0:1], c[:, 4:5], c[:, 1:2], c[:, 5:6])
    mo = rect(c[:, 2:3], c[:, 6:7], c[:, 3:4], c[:, 7:8])
    mb = jnp.maximum(1.0 - ms - mo, 0.0)
    inv = 1.0 / SP
    s_ref[...] = jnp.sum(u * ms, axis=2) * inv
    o_ref[...] = jnp.sum(u * mo, axis=2) * inv
    b_ref[...] = jnp.sum(u * mb, axis=2) * inv


def _fuse_kernel(roi_ref, logits_ref, bboxes_ref, pairs_ref,
                 ss_ref, so_ref, sb_ref,
                 w1a_ref, w1b_ref, w1c_ref, be1_ref, we2_ref, be2_ref,
                 wsr_ref, bsr_ref, wsu_ref, bsu_ref,
                 wor_ref, bor_ref, wou_ref, bou_ref,
                 wbr_ref, bbr_ref, wbu_ref, bbu_ref,
                 wc1a_ref, wc1b_ref, wc1c_ref, bc1_ref, wc2_ref, bc2_ref,
                 wg_ref, bgc_ref, wobj_ref, bobj_ref, wrel_ref, brel_ref,
                 objd_ref, reld_ref):
    f32 = jnp.float32

    def mm(a, b):
        return lax.dot_general(a, b, (((1,), (0,)), ((), ())),
                               preferred_element_type=f32)

    def mm_t(a, b):
        return lax.dot_general(a, b, (((0,), (0,)), ((), ())),
                               preferred_element_type=f32)

    h1 = mm(roi_ref[...], w1a_ref[...]) + mm(logits_ref[...], w1b_ref[...]) \
        + mm(bboxes_ref[...], w1c_ref[...]) + be1_ref[...]
    obj_feats = mm(jnp.maximum(h1, 0.0), we2_ref[...]) + be2_ref[...]

    def gate(s, wr, br, wu, bu):
        a = jax.nn.sigmoid(mm(jnp.maximum(mm(s, wr) + br, 0.0), wu) + bu)
        return s * a

    vs = gate(ss_ref[...], wsr_ref[...], bsr_ref[...], wsu_ref[...], bsu_ref[...])
    vo = gate(so_ref[...], wor_ref[...], bor_ref[...], wou_ref[...], bou_ref[...])
    vb = gate(sb_ref[...], wbr_ref[...], bbr_ref[...], wbu_ref[...], bbu_ref[...])

    rh = jnp.maximum(mm(vs, wc1a_ref[...]) + mm(vo, wc1b_ref[...])
                     + mm(vb, wc1c_ref[...]) + bc1_ref[...], 0.0)
    rel_feats = mm(rh, wc2_ref[...]) + bc2_ref[...]

    n = roi_ref.shape[0]
    m = rel_feats.shape[0]
    pairs = pairs_ref[...]
    obj_ids = lax.broadcasted_iota(jnp.int32, (m, n), 1)
    s_hot = (pairs[:, 0:1] == obj_ids).astype(f32)
    o_hot = (pairs[:, 1:2] == obj_ids).astype(f32)
    so = s_hot + o_hot

    g_obj = mm(obj_feats, wg_ref[...])
    g_rel = mm(rel_feats, wg_ref[...])

    a_oo = mm_t(s_hot, o_hot)
    agg_obj = mm(a_oo, g_obj) + mm_t(so, g_rel) + g_obj
    deg_obj = 1.0 + jnp.sum(a_oo, axis=1, keepdims=True) \
        + jnp.sum(so, axis=0)[:, None]
    h_obj = jnp.maximum(agg_obj / deg_obj + bgc_ref[...], 0.0)

    agg_rel = mm(so, g_obj) + g_rel
    h_rel = jnp.maximum(agg_rel * (1.0 / 3.0) + bgc_ref[...], 0.0)

    out_obj = h_obj + obj_feats
    out_rel = h_rel + rel_feats
    objd_ref[...] = mm(out_obj, wobj_ref[...]) + bobj_ref[...]
    reld_ref[...] = mm(out_rel, wrel_ref[...]) + brel_ref[...]


def kernel(roi_features, obj_logits, bboxes, union_features, rel_pair_idxs,
           We1, be1, We2, be2,
           Wsr, bsr, Wsu, bsu, Wor, bor, Wou, bou, Wbr, bbr, Wbu, bbu,
           Wc1, bc1, Wc2, bc2, Wg, bgc, Wobj, bobj, Wrel, brel):
    f32 = jnp.float32
    n, roi = roi_features.shape
    m, c = union_features.shape[0], union_features.shape[1]
    objc = obj_logits.shape[1]
    relc = Wrel.shape[1]

    sb = bboxes[rel_pair_idxs[:, 0]]
    ob = bboxes[rel_pair_idxs[:, 1]]
    pair_boxes = jnp.concatenate([sb, ob], axis=1)
    union_boxes = jnp.concatenate(
        [jnp.minimum(sb[:, :2], ob[:, :2]), jnp.maximum(sb[:, 2:], ob[:, 2:])], axis=1)
    x = pair_boxes[:, jnp.array([0, 2, 4, 6])] - union_boxes[:, 0:1]
    y = pair_boxes[:, jnp.array([1, 3, 5, 7])] - union_boxes[:, 1:2]
    xr = MS / jnp.maximum(x[:, 1], x[:, 3])
    yr = MS / jnp.maximum(y[:, 1], y[:, 3])
    xp = jnp.clip(jnp.round(x * xr[:, None]), 0, MS)
    yp = jnp.clip(jnp.round(y * yr[:, None]), 0, MS)
    coords = jnp.concatenate([xp, yp], axis=1).astype(f32)

    P = 8
    CB = 128
    grid = (m // P, c // CB)
    uf = union_features.reshape(m, c, SP)
    ss, so, sbg = pl.pallas_call(
        _masked_mean_kernel,
        grid=grid,
        in_specs=[
            pl.BlockSpec((P, 8), lambda i, j: (i, 0)),
            pl.BlockSpec((P, CB, SP), lambda i, j: (i, j, 0)),
        ],
        out_specs=[
            pl.BlockSpec((P, CB), lambda i, j: (i, j)),
            pl.BlockSpec((P, CB), lambda i, j: (i, j)),
            pl.BlockSpec((P, CB), lambda i, j: (i, j)),
        ],
        out_shape=[jax.ShapeDtypeStruct((m, c), f32)] * 3,
    )(coords, uf)

    w1a = We1[:roi]
    w1b = We1[roi:roi + objc]
    w1c = We1[roi + objc:]
    wc1a = Wc1[:c]
    wc1b = Wc1[c:2 * c]
    wc1c = Wc1[2 * c:]
    row = lambda v: v.reshape(1, -1)

    obj_dists, rel_dists = pl.pallas_call(
        _fuse_kernel,
        out_shape=[jax.ShapeDtypeStruct((n, objc), f32),
                   jax.ShapeDtypeStruct((m, relc), f32)],
    )(roi_features, obj_logits, bboxes, rel_pair_idxs,
      ss, so, sbg,
      w1a, w1b, w1c, row(be1), We2, row(be2),
      Wsr, row(bsr), Wsu, row(bsu),
      Wor, row(bor), Wou, row(bou),
      Wbr, row(bbr), Wbu, row(bbu),
      wc1a, wc1b, wc1c, row(bc1), Wc2, row(bc2),
      Wg, row(bgc), Wobj, row(bobj), Wrel, row(brel))
    return (obj_dists, rel_dists)


# speedup vs baseline: 7.7648x; 1.7458x over previous
import jax
import jax.numpy as jnp
from jax import lax
from jax.experimental import pallas as pl
from jax.experimental.pallas import tpu as pltpu

MS = 14
SP = MS * MS


KSLAB = 7
NSTEP = SP // KSLAB


def _masked_mean_kernel(coords_ref, u_ref, s_ref, o_ref, b_ref,
                        acc_s, acc_o, acc_b):
    i = pl.program_id(0)
    c = coords_ref[...]
    x0, x1 = c[:, 0:1], c[:, 1:2]
    ox0, ox1 = c[:, 2:3], c[:, 3:4]
    y0, y1 = c[:, 4:5], c[:, 5:6]
    oy0, oy1 = c[:, 6:7], c[:, 7:8]

    ts = acc_s[...]
    to = acc_o[...]
    tb = acc_b[...]
    zero = jnp.zeros_like(ts)
    if_first = i == 0
    ts = jnp.where(if_first, 0.0, ts)
    to = jnp.where(if_first, 0.0, to)
    tb = jnp.where(if_first, 0.0, tb)

    rf = (i // (MS // KSLAB)).astype(jnp.float32)
    cbase = (i % (MS // KSLAB)) * KSLAB
    rin_s = (rf >= x0) & (rf < x1)
    rin_o = (rf >= ox0) & (rf < ox1)
    for k in range(KSLAB):
        cf = (cbase + k).astype(jnp.float32)
        msk = rin_s & (cf >= y0) & (cf < y1)
        mok = rin_o & (cf >= oy0) & (cf < oy1)
        u = u_ref[0, k]
        ts = ts + jnp.where(msk, u, zero)
        to = to + jnp.where(mok, u, zero)
        tb = tb + jnp.where(msk | mok, zero, u)

    acc_s[...] = ts
    acc_o[...] = to
    acc_b[...] = tb

    @pl.when(i == NSTEP - 1)
    def _():
        inv = 1.0 / SP
        s_ref[...] = ts * inv
        o_ref[...] = to * inv
        b_ref[...] = tb * inv


def _fuse_kernel(roi_ref, logits_ref, bboxes_ref, pairs_ref,
                 ss_ref, so_ref, sb_ref,
                 w1a_ref, w1b_ref, w1c_ref, be1_ref, we2_ref, be2_ref,
                 wsr_ref, bsr_ref, wsu_ref, bsu_ref,
                 wor_ref, bor_ref, wou_ref, bou_ref,
                 wbr_ref, bbr_ref, wbu_ref, bbu_ref,
                 wc1a_ref, wc1b_ref, wc1c_ref, bc1_ref, wc2_ref, bc2_ref,
                 wg_ref, bgc_ref, wobj_ref, bobj_ref, wrel_ref, brel_ref,
                 objd_ref, reld_ref):
    f32 = jnp.float32

    def mm(a, b):
        return lax.dot_general(a, b, (((1,), (0,)), ((), ())),
                               preferred_element_type=f32)

    def mm_t(a, b):
        return lax.dot_general(a, b, (((0,), (0,)), ((), ())),
                               preferred_element_type=f32)

    h1 = mm(roi_ref[...], w1a_ref[...]) + mm(logits_ref[...], w1b_ref[...]) \
        + mm(bboxes_ref[...], w1c_ref[...]) + be1_ref[...]
    obj_feats = mm(jnp.maximum(h1, 0.0), we2_ref[...]) + be2_ref[...]

    def gate(s, wr, br, wu, bu):
        a = jax.nn.sigmoid(mm(jnp.maximum(mm(s, wr) + br, 0.0), wu) + bu)
        return s * a

    vs = gate(ss_ref[...], wsr_ref[...], bsr_ref[...], wsu_ref[...], bsu_ref[...])
    vo = gate(so_ref[...], wor_ref[...], bor_ref[...], wou_ref[...], bou_ref[...])
    vb = gate(sb_ref[...], wbr_ref[...], bbr_ref[...], wbu_ref[...], bbu_ref[...])

    rh = jnp.maximum(mm(vs, wc1a_ref[...]) + mm(vo, wc1b_ref[...])
                     + mm(vb, wc1c_ref[...]) + bc1_ref[...], 0.0)
    rel_feats = mm(rh, wc2_ref[...]) + bc2_ref[...]

    n = roi_ref.shape[0]
    m = rel_feats.shape[0]
    pairs = pairs_ref[...]
    obj_ids = lax.broadcasted_iota(jnp.int32, (m, n), 1)
    s_hot = (pairs[:, 0:1] == obj_ids).astype(f32)
    o_hot = (pairs[:, 1:2] == obj_ids).astype(f32)
    so = s_hot + o_hot

    g_obj = mm(obj_feats, wg_ref[...])
    g_rel = mm(rel_feats, wg_ref[...])

    a_oo = mm_t(s_hot, o_hot)
    agg_obj = mm(a_oo, g_obj) + mm_t(so, g_rel) + g_obj
    deg_obj = 1.0 + jnp.sum(a_oo, axis=1, keepdims=True) \
        + jnp.sum(so, axis=0)[:, None]
    h_obj = jnp.maximum(agg_obj / deg_obj + bgc_ref[...], 0.0)

    agg_rel = mm(so, g_obj) + g_rel
    h_rel = jnp.maximum(agg_rel * (1.0 / 3.0) + bgc_ref[...], 0.0)

    out_obj = h_obj + obj_feats
    out_rel = h_rel + rel_feats
    objd_ref[...] = mm(out_obj, wobj_ref[...]) + bobj_ref[...]
    reld_ref[...] = mm(out_rel, wrel_ref[...]) + brel_ref[...]


def kernel(roi_features, obj_logits, bboxes, union_features, rel_pair_idxs,
           We1, be1, We2, be2,
           Wsr, bsr, Wsu, bsu, Wor, bor, Wou, bou, Wbr, bbr, Wbu, bbu,
           Wc1, bc1, Wc2, bc2, Wg, bgc, Wobj, bobj, Wrel, brel):
    f32 = jnp.float32
    n, roi = roi_features.shape
    m, c = union_features.shape[0], union_features.shape[1]
    objc = obj_logits.shape[1]
    relc = Wrel.shape[1]

    sb = bboxes[rel_pair_idxs[:, 0]]
    ob = bboxes[rel_pair_idxs[:, 1]]
    pair_boxes = jnp.concatenate([sb, ob], axis=1)
    union_boxes = jnp.concatenate(
        [jnp.minimum(sb[:, :2], ob[:, :2]), jnp.maximum(sb[:, 2:], ob[:, 2:])], axis=1)
    x = pair_boxes[:, jnp.array([0, 2, 4, 6])] - union_boxes[:, 0:1]
    y = pair_boxes[:, jnp.array([1, 3, 5, 7])] - union_boxes[:, 1:2]
    xr = MS / jnp.maximum(x[:, 1], x[:, 3])
    yr = MS / jnp.maximum(y[:, 1], y[:, 3])
    xp = jnp.clip(jnp.round(x * xr[:, None]), 0, MS)
    yp = jnp.clip(jnp.round(y * yr[:, None]), 0, MS)
    coords = jnp.concatenate([xp, yp], axis=1).astype(f32)

    ut = jnp.transpose(union_features, (2, 3, 0, 1))
    ss, so, sbg = pl.pallas_call(
        _masked_mean_kernel,
        grid=(NSTEP,),
        in_specs=[
            pl.BlockSpec((m, 8), lambda i: (0, 0)),
            pl.BlockSpec((1, KSLAB, m, c),
                         lambda i: (i // (MS // KSLAB), i % (MS // KSLAB), 0, 0)),
        ],
        out_specs=[
            pl.BlockSpec((m, c), lambda i: (0, 0)),
            pl.BlockSpec((m, c), lambda i: (0, 0)),
            pl.BlockSpec((m, c), lambda i: (0, 0)),
        ],
        out_shape=[jax.ShapeDtypeStruct((m, c), f32)] * 3,
        scratch_shapes=[pltpu.VMEM((m, c), f32)] * 3,
    )(coords, ut)

    w1a = We1[:roi]
    w1b = We1[roi:roi + objc]
    w1c = We1[roi + objc:]
    wc1a = Wc1[:c]
    wc1b = Wc1[c:2 * c]
    wc1c = Wc1[2 * c:]
    row = lambda v: v.reshape(1, -1)

    obj_dists, rel_dists = pl.pallas_call(
        _fuse_kernel,
        out_shape=[jax.ShapeDtypeStruct((n, objc), f32),
                   jax.ShapeDtypeStruct((m, relc), f32)],
    )(roi_features, obj_logits, bboxes, rel_pair_idxs,
      ss, so, sbg,
      w1a, w1b, w1c, row(be1), We2, row(be2),
      Wsr, row(bsr), Wsu, row(bsu),
      Wor, row(bor), Wou, row(bou),
      Wbr, row(bbr), Wbu, row(bbu),
      wc1a, wc1b, wc1c, row(bc1), Wc2, row(bc2),
      Wg, row(bgc), Wobj, row(bobj), Wrel, row(brel))
    return (obj_dists, rel_dists)


# speedup vs baseline: 10.0418x; 1.2932x over previous
import jax
import jax.numpy as jnp
from jax import lax
from jax.experimental import pallas as pl
from jax.experimental.pallas import tpu as pltpu

MS = 14
SP = MS * MS


KSLAB = 7
NSTEP = SP // KSLAB


def _masked_mean_kernel(coords_ref, u_ref, s_ref, o_ref, b_ref,
                        acc_s, acc_o, acc_b):
    i = pl.program_id(0)
    c = coords_ref[...]
    x0, x1 = c[:, 0:1], c[:, 1:2]
    ox0, ox1 = c[:, 2:3], c[:, 3:4]
    y0, y1 = c[:, 4:5], c[:, 5:6]
    oy0, oy1 = c[:, 6:7], c[:, 7:8]

    ts = acc_s[...]
    to = acc_o[...]
    tb = acc_b[...]
    zero = jnp.zeros_like(ts)
    if_first = i == 0
    ts = jnp.where(if_first, 0.0, ts)
    to = jnp.where(if_first, 0.0, to)
    tb = jnp.where(if_first, 0.0, tb)

    rf = (i // (MS // KSLAB)).astype(jnp.float32)
    cbase = (i % (MS // KSLAB)) * KSLAB
    rin_s = (rf >= x0) & (rf < x1)
    rin_o = (rf >= ox0) & (rf < ox1)
    cdim = ts.shape[1]
    sel = (lax.broadcasted_iota(jnp.int32, (3, 3 * cdim), 0)
           == lax.broadcasted_iota(jnp.int32, (3, 3 * cdim), 1) // cdim
           ).astype(jnp.float32)
    for k in range(KSLAB):
        cf = (cbase + k).astype(jnp.float32)
        msk = (rin_s & (cf >= y0) & (cf < y1)).astype(jnp.float32)
        mok = (rin_o & (cf >= oy0) & (cf < oy1)).astype(jnp.float32)
        mbk = jnp.maximum(1.0 - msk - mok, 0.0)
        mvec = jnp.concatenate([msk, mok, mbk], axis=1)
        bc = lax.dot_general(mvec, sel, (((1,), (0,)), ((), ())),
                             preferred_element_type=jnp.float32)
        u = u_ref[0, k]
        ts = ts + u * bc[:, 0:cdim]
        to = to + u * bc[:, cdim:2 * cdim]
        tb = tb + u * bc[:, 2 * cdim:3 * cdim]

    acc_s[...] = ts
    acc_o[...] = to
    acc_b[...] = tb

    @pl.when(i == NSTEP - 1)
    def _():
        inv = 1.0 / SP
        s_ref[...] = ts * inv
        o_ref[...] = to * inv
        b_ref[...] = tb * inv


def _fuse_kernel(roi_ref, logits_ref, bboxes_ref, pairs_ref,
                 ss_ref, so_ref, sb_ref,
                 w1a_ref, w1b_ref, w1c_ref, be1_ref, we2_ref, be2_ref,
                 wsr_ref, bsr_ref, wsu_ref, bsu_ref,
                 wor_ref, bor_ref, wou_ref, bou_ref,
                 wbr_ref, bbr_ref, wbu_ref, bbu_ref,
                 wc1a_ref, wc1b_ref, wc1c_ref, bc1_ref, wc2_ref, bc2_ref,
                 wg_ref, bgc_ref, wobj_ref, bobj_ref, wrel_ref, brel_ref,
                 objd_ref, reld_ref):
    f32 = jnp.float32

    def mm(a, b):
        return lax.dot_general(a, b, (((1,), (0,)), ((), ())),
                               preferred_element_type=f32)

    def mm_t(a, b):
        return lax.dot_general(a, b, (((0,), (0,)), ((), ())),
                               preferred_element_type=f32)

    h1 = mm(roi_ref[...], w1a_ref[...]) + mm(logits_ref[...], w1b_ref[...]) \
        + mm(bboxes_ref[...], w1c_ref[...]) + be1_ref[...]
    obj_feats = mm(jnp.maximum(h1, 0.0), we2_ref[...]) + be2_ref[...]

    def gate(s, wr, br, wu, bu):
        a = jax.nn.sigmoid(mm(jnp.maximum(mm(s, wr) + br, 0.0), wu) + bu)
        return s * a

    vs = gate(ss_ref[...], wsr_ref[...], bsr_ref[...], wsu_ref[...], bsu_ref[...])
    vo = gate(so_ref[...], wor_ref[...], bor_ref[...], wou_ref[...], bou_ref[...])
    vb = gate(sb_ref[...], wbr_ref[...], bbr_ref[...], wbu_ref[...], bbu_ref[...])

    rh = jnp.maximum(mm(vs, wc1a_ref[...]) + mm(vo, wc1b_ref[...])
                     + mm(vb, wc1c_ref[...]) + bc1_ref[...], 0.0)
    rel_feats = mm(rh, wc2_ref[...]) + bc2_ref[...]

    n = roi_ref.shape[0]
    m = rel_feats.shape[0]
    pairs = pairs_ref[...]
    obj_ids = lax.broadcasted_iota(jnp.int32, (m, n), 1)
    s_hot = (pairs[:, 0:1] == obj_ids).astype(f32)
    o_hot = (pairs[:, 1:2] == obj_ids).astype(f32)
    so = s_hot + o_hot

    g_obj = mm(obj_feats, wg_ref[...])
    g_rel = mm(rel_feats, wg_ref[...])

    a_oo = mm_t(s_hot, o_hot)
    agg_obj = mm(a_oo, g_obj) + mm_t(so, g_rel) + g_obj
    deg_obj = 1.0 + jnp.sum(a_oo, axis=1, keepdims=True) \
        + jnp.sum(so, axis=0)[:, None]
    h_obj = jnp.maximum(agg_obj / deg_obj + bgc_ref[...], 0.0)

    agg_rel = mm(so, g_obj) + g_rel
    h_rel = jnp.maximum(agg_rel * (1.0 / 3.0) + bgc_ref[...], 0.0)

    out_obj = h_obj + obj_feats
    out_rel = h_rel + rel_feats
    objd_ref[...] = mm(out_obj, wobj_ref[...]) + bobj_ref[...]
    reld_ref[...] = mm(out_rel, wrel_ref[...]) + brel_ref[...]


def kernel(roi_features, obj_logits, bboxes, union_features, rel_pair_idxs,
           We1, be1, We2, be2,
           Wsr, bsr, Wsu, bsu, Wor, bor, Wou, bou, Wbr, bbr, Wbu, bbu,
           Wc1, bc1, Wc2, bc2, Wg, bgc, Wobj, bobj, Wrel, brel):
    f32 = jnp.float32
    n, roi = roi_features.shape
    m, c = union_features.shape[0], union_features.shape[1]
    objc = obj_logits.shape[1]
    relc = Wrel.shape[1]

    sb = bboxes[rel_pair_idxs[:, 0]]
    ob = bboxes[rel_pair_idxs[:, 1]]
    pair_boxes = jnp.concatenate([sb, ob], axis=1)
    union_boxes = jnp.concatenate(
        [jnp.minimum(sb[:, :2], ob[:, :2]), jnp.maximum(sb[:, 2:], ob[:, 2:])], axis=1)
    x = pair_boxes[:, jnp.array([0, 2, 4, 6])] - union_boxes[:, 0:1]
    y = pair_boxes[:, jnp.array([1, 3, 5, 7])] - union_boxes[:, 1:2]
    xr = MS / jnp.maximum(x[:, 1], x[:, 3])
    yr = MS / jnp.maximum(y[:, 1], y[:, 3])
    xp = jnp.clip(jnp.round(x * xr[:, None]), 0, MS)
    yp = jnp.clip(jnp.round(y * yr[:, None]), 0, MS)
    coords = jnp.concatenate([xp, yp], axis=1).astype(f32)

    ut = jnp.transpose(union_features, (2, 3, 0, 1))
    ss, so, sbg = pl.pallas_call(
        _masked_mean_kernel,
        grid=(NSTEP,),
        in_specs=[
            pl.BlockSpec((m, 8), lambda i: (0, 0)),
            pl.BlockSpec((1, KSLAB, m, c),
                         lambda i: (i // (MS // KSLAB), i % (MS // KSLAB), 0, 0)),
        ],
        out_specs=[
            pl.BlockSpec((m, c), lambda i: (0, 0)),
            pl.BlockSpec((m, c), lambda i: (0, 0)),
            pl.BlockSpec((m, c), lambda i: (0, 0)),
        ],
        out_shape=[jax.ShapeDtypeStruct((m, c), f32)] * 3,
        scratch_shapes=[pltpu.VMEM((m, c), f32)] * 3,
    )(coords, ut)

    w1a = We1[:roi]
    w1b = We1[roi:roi + objc]
    w1c = We1[roi + objc:]
    wc1a = Wc1[:c]
    wc1b = Wc1[c:2 * c]
    wc1c = Wc1[2 * c:]
    row = lambda v: v.reshape(1, -1)

    obj_dists, rel_dists = pl.pallas_call(
        _fuse_kernel,
        out_shape=[jax.ShapeDtypeStruct((n, objc), f32),
                   jax.ShapeDtypeStruct((m, relc), f32)],
    )(roi_features, obj_logits, bboxes, rel_pair_idxs,
      ss, so, sbg,
      w1a, w1b, w1c, row(be1), We2, row(be2),
      Wsr, row(bsr), Wsu, row(bsu),
      Wor, row(bor), Wou, row(bou),
      Wbr, row(bbr), Wbu, row(bbu),
      wc1a, wc1b, wc1c, row(bc1), Wc2, row(bc2),
      Wg, row(bgc), Wobj, row(bobj), Wrel, row(brel))
    return (obj_dists, rel_dists)
